# Initial kernel scaffold; baseline (speedup 1.0000x reference)
import jax
import jax.numpy as jnp
from jax.experimental import pallas as pl


def kernel(premise_indices, hypothesis_indices, W_prem, W_hypo, W_lin, b_lin):
    raise NotImplementedError("write your pallas kernel here")


# speedup vs baseline: 22.9066x; 22.9066x over previous
import functools

import jax
import jax.numpy as jnp
from jax import lax
from jax.experimental import pallas as pl
from jax.experimental.pallas import tpu as pltpu
from jax.experimental.pallas import tpu_sc as plsc

_V = 100000
_E = 64
_B = 16384
_L = 50
_PW = 16

_NC, _NS = 2, 16
_NW = _NC * _NS
_EPW = _B // _NW
_IPW = _EPW * _L
_IC = 100
_IROWS = _IPW // _IC
_CH_IR = 8
_CH_E = _CH_IR * _IC // _L
_NCH = _IROWS // _CH_IR

_VBLK = 2000


def _proj_body(wp_ref, wh_ref, w1_ref, w2_ref, bias_ref, o1_ref, o2_ref):
    o1_ref[...] = (
        jnp.dot(wp_ref[...], w1_ref[...], preferred_element_type=jnp.float32)
        + bias_ref[...]
    )
    o2_ref[...] = (
        jnp.dot(wh_ref[...], w2_ref[...], preferred_element_type=jnp.float32)
        + bias_ref[...]
    )


_proj = pl.pallas_call(
    _proj_body,
    grid=(_V // _VBLK,),
    in_specs=[
        pl.BlockSpec((_VBLK, _E), lambda i: (i, 0)),
        pl.BlockSpec((_VBLK, _E), lambda i: (i, 0)),
        pl.BlockSpec((_E, _PW), lambda i: (0, 0)),
        pl.BlockSpec((_E, _PW), lambda i: (0, 0)),
        pl.BlockSpec((1, _PW), lambda i: (0, 0)),
    ],
    out_specs=[
        pl.BlockSpec((_VBLK, _PW), lambda i: (i, 0)),
        pl.BlockSpec((_VBLK, _PW), lambda i: (i, 0)),
    ],
    out_shape=[
        jax.ShapeDtypeStruct((_V, _PW), jnp.float32),
        jax.ShapeDtypeStruct((_V, _PW), jnp.float32),
    ],
)


def _sc_body(pidx_hbm, hidx_hbm, p1_hbm, p2_hbm, out_hbm,
             pidx_v, hidx_v, pr0, hr0, pr1, hr1, out_v, sem0, sem1):
    wid = lax.axis_index("s") * _NC + lax.axis_index("c")
    irow0 = wid * _IROWS
    pltpu.sync_copy(pidx_hbm.at[pl.ds(irow0, _IROWS)], pidx_v)
    pltpu.sync_copy(hidx_hbm.at[pl.ds(irow0, _IROWS)], hidx_v)

    def fire(c, pr, hr, sem):
        rbase = c * _CH_IR
        for j in range(_CH_IR):
            pltpu.async_copy(p1_hbm.at[pidx_v.at[rbase + j]],
                             pr.at[pl.ds(j * _IC, _IC)], sem)
            pltpu.async_copy(p2_hbm.at[hidx_v.at[rbase + j]],
                             hr.at[pl.ds(j * _IC, _IC)], sem)

    def drain(pr, hr, sem):
        pltpu.make_async_copy(p1_hbm.at[pl.ds(0, _CH_IR * _IC)], pr, sem).wait()
        pltpu.make_async_copy(p2_hbm.at[pl.ds(0, _CH_IR * _IC)], hr, sem).wait()

    def reduce(c, pr, hr):
        ebase = c * _CH_E

        def ebody(e, carry):
            r = e * _L
            a = [pr[r + l, :] for l in range(4)]
            for l in range(4, _L):
                a[l % 4] = a[l % 4] + pr[r + l, :]
            accp = (a[0] + a[1]) + (a[2] + a[3])
            b = [hr[r + l, :] for l in range(4)]
            for l in range(4, _L):
                b[l % 4] = b[l % 4] + hr[r + l, :]
            acch = (b[0] + b[1]) + (b[2] + b[3])
            out_v[ebase + e, :] = accp + acch
            return carry

        lax.fori_loop(0, _CH_E, ebody, 0)

    fire(0, pr0, hr0, sem0)

    def chunk_pair(c2, carry):
        c = c2 * 2
        fire(c + 1, pr1, hr1, sem1)
        drain(pr0, hr0, sem0)
        reduce(c, pr0, hr0)

        @pl.when(c2 < _NCH // 2 - 1)
        def _():
            fire(c + 2, pr0, hr0, sem0)

        drain(pr1, hr1, sem1)
        reduce(c + 1, pr1, hr1)
        return carry

    lax.fori_loop(0, _NCH // 2, chunk_pair, 0)
    pltpu.sync_copy(out_v, out_hbm.at[pl.ds(wid * _EPW, _EPW)])


_sc = functools.partial(
    pl.kernel,
    mesh=plsc.VectorSubcoreMesh(core_axis_name="c", subcore_axis_name="s"),
    out_type=jax.ShapeDtypeStruct((_B, _PW), jnp.float32),
    scratch_types=[
        pltpu.VMEM((_IROWS, _IC), jnp.int32),
        pltpu.VMEM((_IROWS, _IC), jnp.int32),
        pltpu.VMEM((_CH_IR * _IC, _PW), jnp.float32),
        pltpu.VMEM((_CH_IR * _IC, _PW), jnp.float32),
        pltpu.VMEM((_CH_IR * _IC, _PW), jnp.float32),
        pltpu.VMEM((_CH_IR * _IC, _PW), jnp.float32),
        pltpu.VMEM((_EPW, _PW), jnp.float32),
        pltpu.SemaphoreType.DMA,
        pltpu.SemaphoreType.DMA,
    ],
    compiler_params=pltpu.CompilerParams(use_tc_tiling_on_sc=False),
)(_sc_body)


@jax.jit
def kernel(premise_indices, hypothesis_indices, W_prem, W_hypo, W_lin, b_lin):
    pidx2 = premise_indices.astype(jnp.int32).reshape(_B * _L // _IC, _IC)
    hidx2 = hypothesis_indices.astype(jnp.int32).reshape(_B * _L // _IC, _IC)
    w1s = jnp.zeros((_E, _PW), jnp.float32).at[:, :3].set(W_lin[:, :_E].T / _L)
    w2s = jnp.zeros((_E, _PW), jnp.float32).at[:, :3].set(W_lin[:, _E:].T / _L)
    bpad = jnp.zeros((1, _PW), jnp.float32).at[0, :3].set(b_lin / (2 * _L))
    p1, p2 = _proj(W_prem, W_hypo, w1s, w2s, bpad)
    out = _sc(pidx2, hidx2, p1, p2)
    return out[:, :3]


# speedup vs baseline: 38.0244x; 1.6600x over previous
import functools

import jax
import jax.numpy as jnp
from jax import lax
from jax.experimental import pallas as pl
from jax.experimental.pallas import tpu as pltpu
from jax.experimental.pallas import tpu_sc as plsc

_V = 100000
_E = 64
_B = 16384
_L = 50
_PW = 16

_NC, _NS = 2, 16
_NW = _NC * _NS
_EPW = _B // _NW
_IPW = _EPW * _L
_IC = 100
_IROWS = _IPW // _IC
_CH_IR = 8
_CH_E = _CH_IR * _IC // _L
_NCH = _IROWS // _CH_IR

_VBLK = 2048
_VP = 49 * _VBLK


_DN = (((0,), (0,)), ((), ()))
_G = 128 // _PW
_OBLK0 = _VBLK // _G


def _proj_body(wpt_ref, wht_ref, w1_ref, w2_ref, bias_ref, o1_ref, o2_ref):
    wpt = wpt_ref[...]
    wht = wht_ref[...]
    w1 = w1_ref[...]
    w2 = w2_ref[...]
    m1 = jnp.concatenate(
        [lax.dot_general(wpt[:, r * _OBLK0:(r + 1) * _OBLK0], w1, _DN,
                         preferred_element_type=jnp.float32)
         for r in range(_G)], axis=1)
    o1_ref[...] = m1 + bias_ref[...]
    m2 = jnp.concatenate(
        [lax.dot_general(wht[:, r * _OBLK0:(r + 1) * _OBLK0], w2, _DN,
                         preferred_element_type=jnp.float32)
         for r in range(_G)], axis=1)
    o2_ref[...] = m2 + bias_ref[...]


_OBLK = _VBLK * _PW // 128
_PROWS = _VP * _PW // 128

_proj = pl.pallas_call(
    _proj_body,
    grid=(_VP // _VBLK,),
    in_specs=[
        pl.BlockSpec((_E, _VBLK), lambda i: (0, i)),
        pl.BlockSpec((_E, _VBLK), lambda i: (0, i)),
        pl.BlockSpec((_E, _PW), lambda i: (0, 0)),
        pl.BlockSpec((_E, _PW), lambda i: (0, 0)),
        pl.BlockSpec((1, 128), lambda i: (0, 0)),
    ],
    out_specs=[
        pl.BlockSpec((_OBLK, 128), lambda i: (i, 0)),
        pl.BlockSpec((_OBLK, 128), lambda i: (i, 0)),
    ],
    out_shape=[
        jax.ShapeDtypeStruct((_PROWS, 128), jnp.float32),
        jax.ShapeDtypeStruct((_PROWS, 128), jnp.float32),
    ],
)


def _sc_body(pidx_hbm, hidx_hbm, p1_hbm, p2_hbm, out_hbm,
             pidx_v, hidx_v, pr0, hr0, pr1, hr1, out_v, sem0, sem1):
    wid = lax.axis_index("s") * _NC + lax.axis_index("c")
    irow0 = wid * _IROWS
    pltpu.sync_copy(pidx_hbm.at[pl.ds(irow0, _IROWS)], pidx_v)
    pltpu.sync_copy(hidx_hbm.at[pl.ds(irow0, _IROWS)], hidx_v)

    def fire(c, pr, hr, sem):
        rbase = c * _CH_IR
        for j in range(_CH_IR):
            pltpu.async_copy(p1_hbm.at[pidx_v.at[rbase + j]],
                             pr.at[pl.ds(j * _IC, _IC)], sem)
            pltpu.async_copy(p2_hbm.at[hidx_v.at[rbase + j]],
                             hr.at[pl.ds(j * _IC, _IC)], sem)

    def drain(pr, hr, sem):
        pltpu.make_async_copy(p1_hbm.at[pl.ds(0, _CH_IR * _IC)], pr, sem).wait()
        pltpu.make_async_copy(p2_hbm.at[pl.ds(0, _CH_IR * _IC)], hr, sem).wait()

    def reduce(c, pr, hr):
        ebase = c * _CH_E

        def ebody(e, carry):
            r = e * _L
            a = [pr[r + l, :] for l in range(4)]
            for l in range(4, _L):
                a[l % 4] = a[l % 4] + pr[r + l, :]
            accp = (a[0] + a[1]) + (a[2] + a[3])
            b = [hr[r + l, :] for l in range(4)]
            for l in range(4, _L):
                b[l % 4] = b[l % 4] + hr[r + l, :]
            acch = (b[0] + b[1]) + (b[2] + b[3])
            out_v[ebase + e, :] = accp + acch
            return carry

        lax.fori_loop(0, _CH_E, ebody, 0)

    fire(0, pr0, hr0, sem0)

    def chunk_pair(c2, carry):
        c = c2 * 2
        fire(c + 1, pr1, hr1, sem1)
        drain(pr0, hr0, sem0)
        reduce(c, pr0, hr0)

        @pl.when(c2 < _NCH // 2 - 1)
        def _():
            fire(c + 2, pr0, hr0, sem0)

        drain(pr1, hr1, sem1)
        reduce(c + 1, pr1, hr1)
        return carry

    lax.fori_loop(0, _NCH // 2, chunk_pair, 0)
    pltpu.sync_copy(out_v, out_hbm.at[pl.ds(wid * _EPW, _EPW)])


_sc = functools.partial(
    pl.kernel,
    mesh=plsc.VectorSubcoreMesh(core_axis_name="c", subcore_axis_name="s"),
    out_type=jax.ShapeDtypeStruct((_B, _PW), jnp.float32),
    scratch_types=[
        pltpu.VMEM((_IROWS, _IC), jnp.int32),
        pltpu.VMEM((_IROWS, _IC), jnp.int32),
        pltpu.VMEM((_CH_IR * _IC, _PW), jnp.float32),
        pltpu.VMEM((_CH_IR * _IC, _PW), jnp.float32),
        pltpu.VMEM((_CH_IR * _IC, _PW), jnp.float32),
        pltpu.VMEM((_CH_IR * _IC, _PW), jnp.float32),
        pltpu.VMEM((_EPW, _PW), jnp.float32),
        pltpu.SemaphoreType.DMA,
        pltpu.SemaphoreType.DMA,
    ],
    compiler_params=pltpu.CompilerParams(use_tc_tiling_on_sc=False),
)(_sc_body)


@jax.jit
def kernel(premise_indices, hypothesis_indices, W_prem, W_hypo, W_lin, b_lin):
    def perm(v):
        v = v.astype(jnp.int32)
        return (v & ~2047) | ((v & 255) << 3) | ((v >> 8) & 7)

    pidx2 = perm(premise_indices).reshape(_B * _L // _IC, _IC)
    hidx2 = perm(hypothesis_indices).reshape(_B * _L // _IC, _IC)
    w1s = jnp.zeros((_E, _PW), jnp.float32).at[:, :3].set(W_lin[:, :_E].T / _L)
    w2s = jnp.zeros((_E, _PW), jnp.float32).at[:, :3].set(W_lin[:, _E:].T / _L)
    bpad = jnp.tile(
        jnp.zeros((1, _PW), jnp.float32).at[0, :3].set(b_lin / (2 * _L)),
        (1, 128 // _PW))
    p1p, p2p = _proj(W_prem.T, W_hypo.T, w1s, w2s, bpad)
    p1 = p1p.reshape(_VP, _PW)
    p2 = p2p.reshape(_VP, _PW)
    out = _sc(pidx2, hidx2, p1, p2)
    return out[:, :3]
